# baseline (speedup 1.0000x reference)
import functools

import jax
import jax.numpy as jnp
from jax.experimental import pallas as pl
from jax.experimental.pallas import tpu as pltpu

_N = 4096
_D = 16
_K = 15
_BR = 256
_SR = 8


def _body(norm_ref, lat_blk_ref, lat_full_ref, rn_full_ref, dx_ref, mx_ref,
          d12_ref, d21_ref, ov_ref, g_ref):
    i = pl.program_id(0)
    lat_blk = lat_blk_ref[...]
    lat_full = lat_full_ref[...]
    g_ref[...] = jax.lax.dot_general(lat_blk, lat_full, (((1,), (1,)), ((), ())),
                                     preferred_element_type=jnp.float32)
    rn_full = rn_full_ref[...]
    inv_norm = 1.0 / norm_ref[0]

    col = jax.lax.broadcasted_iota(jnp.int32, (_SR, _N), 1)
    srow = jax.lax.broadcasted_iota(jnp.int32, (_SR, _N), 0)

    def strip(s, carry):
        d12, d21, ov = carry
        lat_s = lat_blk_ref[pl.ds(s * _SR, _SR), :]
        rn_s = jnp.sum(lat_s * lat_s, axis=1, keepdims=True)
        gs = g_ref[pl.ds(s * _SR, _SR), :]
        sq = jnp.maximum(rn_s + rn_full - 2.0 * gs, 0.0)
        dz = jnp.sqrt(sq) * inv_norm
        row = srow + (i * _BR + s * _SR)
        on_diag = col == row
        dz = jnp.where(on_diag, 0.0, dz)

        w = dz
        for _ in range(_K + 1):
            mval = jnp.min(w, axis=1, keepdims=True)
            w = jnp.where(w == mval, jnp.inf, w)
        mask_z = jnp.where(jnp.isinf(w) & ~on_diag, 1.0, 0.0)

        dx = dx_ref[pl.ds(s * _SR, _SR), :]
        mx = mx_ref[pl.ds(s * _SR, _SR), :]
        diff = dx - dz
        dsq = diff * diff
        return (d12 + jnp.sum(mx * dsq),
                d21 + jnp.sum(mask_z * dsq),
                ov + jnp.sum(mask_z * mx))

    zero = jnp.float32(0.0)
    d12, d21, ov = jax.lax.fori_loop(0, _BR // _SR, strip, (zero, zero, zero))

    @pl.when(i == 0)
    def _():
        d12_ref[0, 0] = d12
        d21_ref[0, 0] = d21
        ov_ref[0, 0] = ov

    @pl.when(i != 0)
    def _():
        d12_ref[0, 0] += d12
        d21_ref[0, 0] += d21
        ov_ref[0, 0] += ov


@jax.jit
def kernel(latent, latent_norm, dist_X, pair_mask_X):
    n, k = _N, _K
    rn_full = jnp.sum(latent * latent, axis=1)[None, :]
    norm = latent_norm.reshape((1,))
    grid = (n // _BR,)
    scalar_spec = pl.BlockSpec(memory_space=pltpu.SMEM)
    out = pl.pallas_call(
        _body,
        grid=grid,
        in_specs=[
            scalar_spec,
            pl.BlockSpec((_BR, _D), lambda i: (i, 0)),
            pl.BlockSpec((_N, _D), lambda i: (0, 0)),
            pl.BlockSpec((1, _N), lambda i: (0, 0)),
            pl.BlockSpec((_BR, _N), lambda i: (i, 0)),
            pl.BlockSpec((_BR, _N), lambda i: (i, 0)),
        ],
        out_specs=[
            pl.BlockSpec((1, 1), lambda i: (0, 0), memory_space=pltpu.SMEM),
            pl.BlockSpec((1, 1), lambda i: (0, 0), memory_space=pltpu.SMEM),
            pl.BlockSpec((1, 1), lambda i: (0, 0), memory_space=pltpu.SMEM),
        ],
        out_shape=[
            jax.ShapeDtypeStruct((1, 1), jnp.float32),
            jax.ShapeDtypeStruct((1, 1), jnp.float32),
            jax.ShapeDtypeStruct((1, 1), jnp.float32),
        ],
        scratch_shapes=[pltpu.VMEM((_BR, _N), jnp.float32)],
    )(norm, latent, latent, rn_full, dist_X, pair_mask_X)
    d12 = out[0][0, 0]
    d21 = out[1][0, 0]
    ov = out[2][0, 0]
    distance = d12 + d21
    matched_pairs = ov / (n * k)
    return (distance, matched_pairs, d12, d21)


# speedup vs baseline: 3.8838x; 3.8838x over previous
import functools

import jax
import jax.numpy as jnp
from jax.experimental import pallas as pl
from jax.experimental.pallas import tpu as pltpu

_N = 4096
_D = 16
_K = 15
_BR = 256


def _body(norm_ref, lat_blk_ref, lat_full_ref, rn_full_ref, dx_ref, mx_ref,
          d12_ref, d21_ref, ov_ref):
    i = pl.program_id(0)
    lat_blk = lat_blk_ref[...]
    lat_full = lat_full_ref[...]
    g = jax.lax.dot_general(lat_blk, lat_full, (((1,), (1,)), ((), ())),
                            preferred_element_type=jnp.float32,
                            precision=jax.lax.Precision.HIGHEST)
    rn_blk = jnp.sum(lat_blk * lat_blk, axis=1, keepdims=True)
    rn_full = rn_full_ref[...]
    sq = jnp.maximum(rn_blk + rn_full - 2.0 * g, 0.0)
    inv_norm = 1.0 / norm_ref[0]
    dz = jnp.sqrt(sq) * inv_norm
    col = jax.lax.broadcasted_iota(jnp.int32, (_BR, _N), 1)
    row = jax.lax.broadcasted_iota(jnp.int32, (_BR, _N), 0) + i * _BR
    on_diag = col == row
    dz = jnp.where(on_diag, 0.0, dz)

    w = dz
    for _ in range(_K + 1):
        mval = jnp.min(w, axis=1, keepdims=True)
        w = jnp.where(w == mval, jnp.inf, w)
    mask_z = jnp.where(jnp.isinf(w) & ~on_diag, 1.0, 0.0)

    dx = dx_ref[...]
    mx = mx_ref[...]
    diff = dx - dz
    dsq = diff * diff
    d12 = jnp.sum(mx * dsq)
    d21 = jnp.sum(mask_z * dsq)
    ov = jnp.sum(mask_z * mx)

    @pl.when(i == 0)
    def _():
        d12_ref[0, 0] = d12
        d21_ref[0, 0] = d21
        ov_ref[0, 0] = ov

    @pl.when(i != 0)
    def _():
        d12_ref[0, 0] += d12
        d21_ref[0, 0] += d21
        ov_ref[0, 0] += ov


@jax.jit
def kernel(latent, latent_norm, dist_X, pair_mask_X):
    n, k = _N, _K
    rn_full = jnp.sum(latent * latent, axis=1)[None, :]
    norm = latent_norm.reshape((1,))
    grid = (n // _BR,)
    scalar_spec = pl.BlockSpec(memory_space=pltpu.SMEM)
    out = pl.pallas_call(
        _body,
        grid=grid,
        in_specs=[
            scalar_spec,
            pl.BlockSpec((_BR, _D), lambda i: (i, 0)),
            pl.BlockSpec((_N, _D), lambda i: (0, 0)),
            pl.BlockSpec((1, _N), lambda i: (0, 0)),
            pl.BlockSpec((_BR, _N), lambda i: (i, 0)),
            pl.BlockSpec((_BR, _N), lambda i: (i, 0)),
        ],
        out_specs=[
            pl.BlockSpec((1, 1), lambda i: (0, 0), memory_space=pltpu.SMEM),
            pl.BlockSpec((1, 1), lambda i: (0, 0), memory_space=pltpu.SMEM),
            pl.BlockSpec((1, 1), lambda i: (0, 0), memory_space=pltpu.SMEM),
        ],
        out_shape=[
            jax.ShapeDtypeStruct((1, 1), jnp.float32),
            jax.ShapeDtypeStruct((1, 1), jnp.float32),
            jax.ShapeDtypeStruct((1, 1), jnp.float32),
        ],
    )(norm, latent, latent, rn_full, dist_X, pair_mask_X)
    d12 = out[0][0, 0]
    d21 = out[1][0, 0]
    ov = out[2][0, 0]
    distance = d12 + d21
    matched_pairs = ov / (n * k)
    return (distance, matched_pairs, d12, d21)
